# Initial kernel scaffold; baseline (speedup 1.0000x reference)
import jax
import jax.numpy as jnp
from jax.experimental import pallas as pl


def kernel(gcn_w, gcn_b, prelu_alpha, bil_w, bil_b, x, adj, idx, subgraphs):
    raise NotImplementedError("write your pallas kernel here")


# speedup vs baseline: 1.2993x; 1.2993x over previous
import numpy as np
import jax
import jax.numpy as jnp
from jax.experimental import pallas as pl
from jax.experimental.pallas import tpu as pltpu


def _precompute_kernel(x_ref, w_ref, b_ref, alpha_ref, bw_ref, xw_ref, ha_ref):
    xw = jnp.dot(x_ref[...], w_ref[...], preferred_element_type=jnp.float32)
    xw_ref[...] = xw
    h = xw + b_ref[...]
    alpha = alpha_ref[0, 0]
    h = jnp.where(h >= 0.0, h, alpha * h)
    ha_ref[...] = jnp.dot(h, bw_ref[...], preferred_element_type=jnp.float32)


def _agg_score_kernel(gt_ref, a_ref, ha_ref, ab_ref, gb_ref, b_ref, alpha_ref,
                      bb_ref, o_ref):
    bias = b_ref[...]
    alpha = alpha_ref[0, 0]
    bb = bb_ref[0, 0]
    a = a_ref[...]
    g = [gt_ref[t] for t in range(3)]

    hs = []
    for r in range(4):
        acc = bias + a[:, 3 * r:3 * r + 1] * g[0]
        acc = acc + a[:, 3 * r + 1:3 * r + 2] * g[1]
        acc = acc + a[:, 3 * r + 2:3 * r + 3] * g[2]
        hs.append(jnp.where(acc >= 0.0, acc, alpha * acc))
    c = (hs[0] + hs[1] + hs[2] + hs[3]) * 0.25

    ha = ha_ref[...]
    pos = jnp.sum(ha * c, axis=1, keepdims=True) + bb

    ab = ab_ref[0]
    gb = gb_ref[0]
    hbs = []
    for r in range(4):
        accb = bias + ab[:, 3 * r:3 * r + 1] * gb[0:1]
        accb = accb + ab[:, 3 * r + 1:3 * r + 2] * gb[1:2]
        accb = accb + ab[:, 3 * r + 2:3 * r + 3] * gb[2:3]
        hbs.append(jnp.where(accb >= 0.0, accb, alpha * accb))
    cb = (hbs[0] + hbs[1] + hbs[2] + hbs[3]) * 0.25

    c_shift = jnp.concatenate([cb, c[:-1]], axis=0)
    neg = jnp.sum(ha * c_shift, axis=1, keepdims=True) + bb
    o_ref[...] = jnp.concatenate([pos, neg], axis=1)


def _forward(gcn_w, gcn_b, prelu_alpha, bil_w, bil_b, x, adj, idx, subgraphs,
             bt=256, blk_a=1024):
    B = idx.shape[0]
    N, n_in = x.shape[1], x.shape[2]
    n_h = gcn_w.shape[1]
    nblk = B // bt
    bias = gcn_b.reshape(1, n_h)

    xw, ha_all = pl.pallas_call(
        _precompute_kernel,
        out_shape=(jax.ShapeDtypeStruct((N, n_h), jnp.float32),
                   jax.ShapeDtypeStruct((N, n_h), jnp.float32)),
        grid=(N // blk_a,),
        in_specs=[
            pl.BlockSpec((blk_a, n_in), lambda i: (i, 0)),
            pl.BlockSpec((n_in, n_h), lambda i: (0, 0)),
            pl.BlockSpec((1, n_h), lambda i: (0, 0)),
            pl.BlockSpec(memory_space=pltpu.MemorySpace.SMEM),
            pl.BlockSpec((n_h, n_h), lambda i: (0, 0)),
        ],
        out_specs=(pl.BlockSpec((blk_a, n_h), lambda i: (i, 0)),
                   pl.BlockSpec((blk_a, n_h), lambda i: (i, 0))),
        compiler_params=pltpu.CompilerParams(dimension_semantics=("parallel",)),
    )(x[0], gcn_w, bias, prelu_alpha, bil_w)

    sub = subgraphs[idx]
    subc = sub[:, :3]
    gt = xw[subc.T]
    hag = ha_all[sub[:, 3]]
    a12 = adj[0][sub[:, :, None], subc[:, None, :]].reshape(B, 12)

    bidx = np.concatenate([[B - 2], np.arange(1, nblk) * bt - 1])
    ab = a12[bidx].reshape(nblk, 1, 12)
    gb = jnp.transpose(gt[:, bidx, :], (1, 0, 2))

    scores = pl.pallas_call(
        _agg_score_kernel,
        out_shape=jax.ShapeDtypeStruct((B, 2), jnp.float32),
        grid=(nblk,),
        in_specs=[
            pl.BlockSpec((3, bt, n_h), lambda i: (0, i, 0)),
            pl.BlockSpec((bt, 12), lambda i: (i, 0)),
            pl.BlockSpec((bt, n_h), lambda i: (i, 0)),
            pl.BlockSpec((1, 1, 12), lambda i: (i, 0, 0)),
            pl.BlockSpec((1, 3, n_h), lambda i: (i, 0, 0)),
            pl.BlockSpec((1, n_h), lambda i: (0, 0)),
            pl.BlockSpec(memory_space=pltpu.MemorySpace.SMEM),
            pl.BlockSpec(memory_space=pltpu.MemorySpace.SMEM),
        ],
        out_specs=pl.BlockSpec((bt, 2), lambda i: (i, 0)),
        compiler_params=pltpu.CompilerParams(dimension_semantics=("parallel",)),
    )(gt, a12, hag, ab, gb, bias, prelu_alpha, bil_b)

    return scores.T.reshape(-1, 1)


def kernel(gcn_w, gcn_b, prelu_alpha, bil_w, bil_b, x, adj, idx, subgraphs):
    return _forward(gcn_w, gcn_b, prelu_alpha, bil_w, bil_b, x, adj, idx,
                    subgraphs)
